# baseline (speedup 1.0000x reference)
import functools

import jax
import jax.numpy as jnp
from jax import lax
from jax.experimental import pallas as pl
from jax.experimental.pallas import tpu as pltpu
from jax.experimental.pallas import tpu_sc as plsc

TOPK = 5
BATCH_N = 128
VOCAB_N = 100000
LANES = 16
NCORES = 2
NSUB = 16
NLAB = 20
NBG = BATCH_N // LANES
BV = 2000
NBLK = VOCAB_N // BV

_sc_params = pltpu.CompilerParams(
    needs_layout_passes=False, use_tc_tiling_on_sc=False)
_mesh = plsc.VectorSubcoreMesh(core_axis_name="c", subcore_axis_name="s")


@functools.partial(
    pl.kernel,
    out_type=jax.ShapeDtypeStruct((BATCH_N,), jnp.float32),
    mesh=_mesh,
    compiler_params=_sc_params,
    scratch_types=[
        pltpu.VMEM((NLAB, BATCH_N), jnp.int32),
        pltpu.VMEM((NLAB, BATCH_N), jnp.int32),
        pltpu.VMEM((NLAB, BATCH_N), jnp.float32),
        pltpu.VMEM((BATCH_N,), jnp.float32),
        pltpu.SemaphoreType.DMA,
    ],
)
def _sc_lmax(pflat_hbm, labt_hbm, out_hbm, labv, idxv, lval, lm_v, gsem):
    core = lax.axis_index("c")
    sid = lax.axis_index("s")
    iota = lax.iota(jnp.int32, LANES)

    @pl.when((core == 0) & (sid == 0))
    def _():
        pltpu.sync_copy(labt_hbm, labv)
        for j in range(NLAB):
            for bg in range(NBG):
                lab = labv[j, pl.ds(bg * LANES, LANES)]
                idxv[j, pl.ds(bg * LANES, LANES)] = (
                    lab * BATCH_N + (bg * LANES + iota))
        gathers = [
            pltpu.make_async_copy(pflat_hbm.at[idxv.at[j]], lval.at[j], gsem)
            for j in range(NLAB)
        ]
        for g in gathers:
            g.start()
        for g in gathers:
            g.wait()
        for bg in range(NBG):
            m = lval[0, pl.ds(bg * LANES, LANES)]
            for j in range(1, NLAB):
                m = jnp.maximum(m, lval[j, pl.ds(bg * LANES, LANES)])
            lm_v[pl.ds(bg * LANES, LANES)] = m
        pltpu.sync_copy(lm_v, out_hbm)


def _tc_count_body(pt_ref, lmax_ref, out_ref, cnt_ref):
    i = pl.program_id(0)
    x = pt_ref[...]
    m = lmax_ref[...]
    c = jnp.sum((x > m).astype(jnp.float32), axis=0, keepdims=True)

    @pl.when(i == 0)
    def _():
        cnt_ref[...] = c

    @pl.when(i > 0)
    def _():
        cnt_ref[...] = cnt_ref[...] + c

    @pl.when(i == NBLK - 1)
    def _():
        tot = cnt_ref[...]
        matches = jnp.sum(
            jnp.where(tot < TOPK, jnp.float32(1.0), jnp.float32(0.0)))
        out_ref[...] = jnp.broadcast_to(matches, (1, 1))


_tc_count = pl.pallas_call(
    _tc_count_body,
    grid=(NBLK,),
    in_specs=[
        pl.BlockSpec((BV, BATCH_N), lambda i: (i, 0)),
        pl.BlockSpec((1, BATCH_N), lambda i: (0, 0)),
    ],
    out_specs=pl.BlockSpec((1, 1), lambda i: (0, 0)),
    out_shape=jax.ShapeDtypeStruct((1, 1), jnp.float32),
    scratch_shapes=[pltpu.VMEM((1, BATCH_N), jnp.float32)],
    compiler_params=pltpu.CompilerParams(
        dimension_semantics=("arbitrary",)),
)


def kernel(probas, labels):
    pt = probas.T
    pflat = pt.reshape(-1)
    labt = labels.T
    lmax = _sc_lmax(pflat, labt)
    matches = _tc_count(pt, lmax.reshape(1, BATCH_N))
    return matches[0, 0] * jnp.float32(1.0 / BATCH_N)


# speedup vs baseline: 1.4570x; 1.4570x over previous
import functools

import jax
import jax.numpy as jnp
from jax import lax
from jax.experimental import pallas as pl
from jax.experimental.pallas import tpu as pltpu
from jax.experimental.pallas import tpu_sc as plsc

TOPK = 5
BATCH_N = 128
VOCAB_N = 100000
LANES = 16
NCORES = 2
NSUB = 16
NLAB = 20
BHALF = BATCH_N // NCORES
NBG = BHALF // LANES
NQ = BATCH_N // LANES
VSLAB = VOCAB_N // NSUB
CH_ROWS = 625
NCHUNK = VSLAB // CH_ROWS
UNROLL = 25
INNER = CH_ROWS // UNROLL
LROWS = 2
NGTILES = NLAB // LROWS

_params = pltpu.CompilerParams(
    needs_layout_passes=False, use_tc_tiling_on_sc=False)
_mesh = plsc.VectorSubcoreMesh(core_axis_name="c", subcore_axis_name="s")


@functools.partial(
    pl.kernel,
    out_type=jax.ShapeDtypeStruct((NCORES, LANES), jnp.float32),
    mesh=_mesh,
    compiler_params=_params,
    scratch_types=[
        pltpu.VMEM((2, CH_ROWS, BHALF), jnp.float32),
        pltpu.VMEM((NLAB, BATCH_N), jnp.int32),
        pltpu.VMEM((LROWS, BATCH_N), jnp.int32),
        pltpu.VMEM((LROWS, BATCH_N, BATCH_N), jnp.float32),
        pltpu.VMEM((BHALF,), jnp.float32),
        pltpu.VMEM((NSUB, BHALF), jnp.float32),
        pltpu.VMEM((NBG, LANES), jnp.float32),
        pltpu.VMEM((BHALF,), jnp.int32),
        pltpu.VMEM((NSUB, BHALF), jnp.int32),
        pltpu.VMEM((LANES,), jnp.float32),
        pltpu.VMEM_SHARED((NSUB, BHALF), jnp.float32),
        pltpu.VMEM_SHARED((NSUB, BHALF), jnp.int32),
        pltpu.SemaphoreType.DMA,
        pltpu.SemaphoreType.DMA,
        pltpu.SemaphoreType.DMA,
    ],
)
def _sc_topk_acc(pt_hbm, labt_hbm, out_hbm,
                 buf, labv, idxg, grow, plm_v, slm_v, lmax_v,
                 cnt_v, sums_v, outv, shared_l, shared_c,
                 sem0, sem1, gsem):
    core = lax.axis_index("c")
    sid = lax.axis_index("s")
    sems = (sem0, sem1)
    iota = lax.iota(jnp.int32, LANES)

    row0 = sid * VSLAB
    col0 = core * BHALF
    q0 = core * NBG

    def chunk_cp(c):
        return pltpu.make_async_copy(
            pt_hbm.at[pl.ds(row0 + c * CH_ROWS, CH_ROWS),
                      pl.ds(col0, BHALF)],
            buf.at[c & 1],
            sems[c & 1],
        )

    chunk_cp(0).start()
    chunk_cp(1).start()

    pltpu.sync_copy(labt_hbm, labv)
    neg = jnp.full((LANES,), -jnp.inf, jnp.float32)
    j0 = sid * LROWS
    gactive = sid < NGTILES

    @pl.when(gactive)
    def _():
        for t in range(LROWS):
            for blk in range(NQ):
                idxg[t, pl.ds(blk * LANES, LANES)] = (
                    labv[j0 + t, pl.ds(blk * LANES, LANES)])
        gathers = [
            pltpu.make_async_copy(pt_hbm.at[idxg.at[t]], grow.at[t], gsem)
            for t in range(LROWS)
        ]
        for g in gathers:
            g.start()
        for g in gathers:
            g.wait()
        for bg in range(NBG):
            m = neg
            bv = col0 + bg * LANES + iota
            for t in range(LROWS):
                tv = jnp.broadcast_to(t, (LANES,)).astype(jnp.int32)
                vals = plsc.load_gather(grow, [tv, bv, bv])
                m = jnp.maximum(m, vals)
            plm_v[pl.ds(bg * LANES, LANES)] = m

    @pl.when(jnp.logical_not(gactive))
    def _():
        for bg in range(NBG):
            plm_v[pl.ds(bg * LANES, LANES)] = neg

    pltpu.sync_copy(plm_v, shared_l.at[sid])
    plsc.subcore_barrier()
    pltpu.sync_copy(shared_l, slm_v)
    for bg in range(NBG):
        m = slm_v[0, pl.ds(bg * LANES, LANES)]
        for r in range(1, NSUB):
            m = jnp.maximum(m, slm_v[r, pl.ds(bg * LANES, LANES)])
        lmax_v[bg] = m

    zero = jnp.zeros((LANES,), jnp.int32)
    ones = jnp.ones((LANES,), jnp.int32)
    for bg in range(NBG):
        cnt_v[pl.ds(bg * LANES, LANES)] = zero

    for c in range(NCHUNK):
        chunk_cp(c).wait()
        if c + 2 < NCHUNK:
            chunk_cp(c + 2).start()
        cb = c & 1

        def bg_body(bg, _, cb=cb):
            lmax = lmax_v[bg]
            boff = bg * LANES

            def row_body(i, accs, cb=cb, boff=boff, lmax=lmax):
                out = []
                for k in range(UNROLL):
                    v = buf[cb, i * UNROLL + k, pl.ds(boff, LANES)]
                    out.append(accs[k] + jnp.where(v > lmax, ones, zero))
                return tuple(out)

            accs = list(lax.fori_loop(0, INNER, row_body, (zero,) * UNROLL))
            while len(accs) > 1:
                nxt = [accs[i] + accs[i + 1]
                       for i in range(0, len(accs) - 1, 2)]
                if len(accs) % 2:
                    nxt.append(accs[-1])
                accs = nxt
            cnt_v[pl.ds(boff, LANES)] = cnt_v[pl.ds(boff, LANES)] + accs[0]
            return 0

        lax.fori_loop(0, NBG, bg_body, 0)

    pltpu.sync_copy(cnt_v, shared_c.at[sid])
    plsc.subcore_barrier()

    @pl.when(sid == 0)
    def _():
        pltpu.sync_copy(shared_c, sums_v)
        nmatch = jnp.float32(0.0)
        for bg in range(NBG):
            tot = sums_v[0, pl.ds(bg * LANES, LANES)]
            for r in range(1, NSUB):
                tot = tot + sums_v[r, pl.ds(bg * LANES, LANES)]
            nmatch = nmatch + jnp.sum(
                jnp.where(tot < TOPK, jnp.float32(1.0), jnp.float32(0.0)))
        outv[...] = jnp.broadcast_to(nmatch, (LANES,))
        pltpu.sync_copy(outv, out_hbm.at[core])


def kernel(probas, labels):
    pt = probas.T
    labt = labels.T
    out = _sc_topk_acc(pt, labt)
    return (out[0, 0] + out[1, 0]) * jnp.float32(1.0 / BATCH_N)
